# Initial kernel scaffold; baseline (speedup 1.0000x reference)
import jax
import jax.numpy as jnp
from jax.experimental import pallas as pl


def kernel(X, NX, EW, node_emb, edge_w, node_w, fc_W, fc_b):
    raise NotImplementedError("write your pallas kernel here")


# speedup vs baseline: 2.4997x; 2.4997x over previous
import functools

import jax
import jax.numpy as jnp
from jax import lax
from jax.experimental import pallas as pl
from jax.experimental.pallas import tpu as pltpu
from jax.experimental.pallas import tpu_sc as plsc

_NUM_NODE = 5000
_EMBED = 128
_NUM_CLS = 20
_BZ, _SEQ, _NBR = 1024, 50, 8
_PAIRS = _SEQ * _NBR
_SEQ_PAD = 56
_LANES = 16
_NVEC = _EMBED // _LANES

_info = plsc.get_sparse_core_info()
_NC, _NS = _info.num_cores, _info.num_subcores
_NW = _NC * _NS
_B_PER_W = _BZ // _NW

_CHUNKS = ((0, 128), (128, 128), (256, 128), (384, 16))


@functools.partial(
    pl.kernel,
    out_type=jax.ShapeDtypeStruct((_BZ, _EMBED), jnp.float32),
    mesh=plsc.VectorSubcoreMesh(core_axis_name="c", subcore_axis_name="s"),
    scratch_types=[
        pltpu.VMEM((_PAIRS,), jnp.int32),
        pltpu.VMEM((_PAIRS,), jnp.int32),
        pltpu.VMEM((_SEQ_PAD,), jnp.int32),
        pltpu.VMEM((_PAIRS, _EMBED), jnp.float32),
        pltpu.VMEM((_PAIRS + _LANES,), jnp.float32),
        pltpu.VMEM((_SEQ_PAD, _EMBED), jnp.float32),
        pltpu.VMEM((_SEQ_PAD + _LANES,), jnp.float32),
        pltpu.VMEM((_EMBED,), jnp.float32),
        pltpu.SemaphoreType.DMA,
    ],
)
def _sc_pool(emb_hbm, ew_hbm, nw_hbm, x_hbm, nx_hbm, ewi_hbm, out_hbm,
             nx_idx, ew_idx, x_idx, rows, ewv, rn, nn, hst, sem):
    wid = lax.axis_index("s") * _NC + lax.axis_index("c")

    def b_body(lb, carry):
        b = wid * _B_PER_W + lb
        pltpu.sync_copy(nx_hbm.at[b], nx_idx)
        pltpu.sync_copy(ewi_hbm.at[b], ew_idx)
        pltpu.sync_copy(x_hbm.at[b], x_idx)
        copies = []
        for off, sz in _CHUNKS:
            copies.append(pltpu.async_copy(
                emb_hbm.at[nx_idx.at[pl.ds(off, sz)]],
                rows.at[pl.ds(off, sz)], sem))
            copies.append(pltpu.async_copy(
                ew_hbm.at[ew_idx.at[pl.ds(off, sz)]],
                ewv.at[pl.ds(off, sz)], sem))
        copies.append(pltpu.async_copy(emb_hbm.at[x_idx], rn, sem))
        copies.append(pltpu.async_copy(
            nw_hbm.at[x_idx], nn.at[pl.ds(0, _SEQ_PAD)], sem))
        for c in copies:
            c.wait()

        def s_body(s, acc):
            base = s * _NBR
            wv = ewv[pl.ds(base, _LANES)]
            m = [None] * _NVEC
            for n in range(_NBR):
                wb = jnp.full((_LANES,), wv[n], jnp.float32)
                for e in range(_NVEC):
                    v = rows[base + n, pl.ds(e * _LANES, _LANES)] * wb
                    m[e] = v if n == 0 else jnp.maximum(m[e], v)
            nb = jnp.full((_LANES,), nn[pl.ds(s, _LANES)][0], jnp.float32)
            ob = 1.0 - nb
            return tuple(
                acc[e] + ob * m[e] + nb * rn[s, pl.ds(e * _LANES, _LANES)]
                for e in range(_NVEC))

        acc0 = tuple(jnp.zeros((_LANES,), jnp.float32) for _ in range(_NVEC))
        acc = lax.fori_loop(0, _SEQ, s_body, acc0)
        for e in range(_NVEC):
            hst[pl.ds(e * _LANES, _LANES)] = acc[e]
        pltpu.sync_copy(hst, out_hbm.at[b])
        return carry

    lax.fori_loop(0, _B_PER_W, b_body, 0)


def _fc_body(h_ref, w_ref, b_ref, o_ref):
    z = jnp.dot(h_ref[...], w_ref[...], preferred_element_type=jnp.float32)
    z = jnp.maximum(z + b_ref[...], 0.0)
    mx = jnp.max(z, axis=1, keepdims=True)
    ez = jnp.exp(z - mx)
    lse = jnp.log(jnp.sum(ez, axis=1, keepdims=True)) + mx
    o_ref[...] = z - lse


def kernel(X, NX, EW, node_emb, edge_w, node_w, fc_W, fc_b):
    X = X.astype(jnp.int32)
    NX = NX.astype(jnp.int32)
    EW = EW.astype(jnp.int32)
    x_pad = jnp.pad(X, ((0, 0), (0, _SEQ_PAD - _SEQ)))
    nx_flat = NX.reshape(_BZ, _PAIRS)
    ew_flat = EW.reshape(_BZ, _PAIRS)
    h = _sc_pool(node_emb, edge_w.reshape(-1), node_w.reshape(-1),
                 x_pad, nx_flat, ew_flat)
    return pl.pallas_call(
        _fc_body,
        out_shape=jax.ShapeDtypeStruct((_BZ, _NUM_CLS), jnp.float32),
    )(h, fc_W, fc_b.reshape(1, _NUM_CLS))
